# Initial kernel scaffold; baseline (speedup 1.0000x reference)
import jax
import jax.numpy as jnp
from jax.experimental import pallas as pl


def kernel(data, pts, node_keys, depth):
    raise NotImplementedError("write your pallas kernel here")


# speedup vs baseline: 55.8253x; 55.8253x over previous
import functools

import jax
import jax.numpy as jnp
from jax import lax
from jax.experimental import pallas as pl
from jax.experimental.pallas import tpu as pltpu
from jax.experimental.pallas import tpu_sc as plsc

NC = 2
NS = 16
NW = NC * NS
L = 16

DEPTH_C = 6
KEY_SPACE = 1 << (3 * DEPTH_C)
GRID = 1 << DEPTH_C
LUT_SLICE = 8224
LUT_SIZE = NW * LUT_SLICE
SENTINEL_KEY = KEY_SPACE
NK_WIN = LUT_SLICE + 16
PAD_KEY = 2 ** 30

SB = 256
NIDX = 8 * SB
DMA_I = 128
C = 32


def _interleave(v, s):
    k = jnp.zeros_like(v)
    for i in range(DEPTH_C):
        k = k | ((v & (1 << i)) << (2 * i + s))
    return k


def _floor_i32(xf):
    xt = xf.astype(jnp.int32)
    xi = jnp.where(xt.astype(jnp.float32) > xf, xt - 1, xt)
    return xi


def _build_lut(node_keys_pad, starts8, sentinel_node):
    mesh = plsc.VectorSubcoreMesh(core_axis_name="c", subcore_axis_name="s")

    @functools.partial(
        pl.kernel,
        out_type=jax.ShapeDtypeStruct((LUT_SIZE,), jnp.int32),
        mesh=mesh,
        compiler_params=pltpu.CompilerParams(
            needs_layout_passes=False, use_tc_tiling_on_sc=False),
        scratch_types=[
            pltpu.VMEM((NK_WIN,), jnp.int32),
            pltpu.VMEM((LUT_SLICE,), jnp.int32),
            pltpu.VMEM((NW + L,), jnp.int32),
        ],
    )
    def lut_kernel(nk_hbm, st_hbm, lut_hbm, nk_v, lut_v, st_v):
        wid = lax.axis_index("s") * NC + lax.axis_index("c")
        pltpu.sync_copy(st_hbm, st_v)
        s8 = pl.multiple_of(st_v[pl.ds(wid, L)][0], 8)
        pltpu.sync_copy(nk_hbm.at[pl.ds(s8, NK_WIN)], nk_v)

        fill = jnp.full((L,), sentinel_node, dtype=jnp.int32)

        def init_body(t, _):
            lut_v[pl.ds(t * L, L)] = fill
            return 0

        lax.fori_loop(0, LUT_SLICE // L, init_body, 0)

        base_local = wid * LUT_SLICE
        lanes = lax.iota(jnp.int32, L)

        def scat_body(t, _):
            kv = nk_v[pl.ds(t * L, L)]
            local = kv - base_local
            mask = (local >= 0) & (local < LUT_SLICE)
            localc = jnp.clip(local, 0, LUT_SLICE - 1)
            gi = s8 + t * L + lanes
            plsc.store_scatter(lut_v, [localc], gi, mask=mask)
            return 0

        lax.fori_loop(0, NK_WIN // L, scat_body, 0)
        pltpu.sync_copy(lut_v, lut_hbm.at[pl.ds(wid * LUT_SLICE, LUT_SLICE)])

    return lut_kernel(node_keys_pad, starts8)


def _interp(xp, yp, zp, lut, data_aug, npw, sentinel_node):
    mesh = plsc.VectorSubcoreMesh(core_axis_name="c", subcore_axis_name="s")
    np_total = npw * NW
    nb = npw // SB

    @functools.partial(
        pl.kernel,
        out_type=jax.ShapeDtypeStruct((np_total, C), jnp.float32),
        mesh=mesh,
        compiler_params=pltpu.CompilerParams(
            needs_layout_passes=False, use_tc_tiling_on_sc=False),
        scratch_types=[
            pltpu.VMEM((SB,), jnp.float32),
            pltpu.VMEM((SB,), jnp.float32),
            pltpu.VMEM((SB,), jnp.float32),
            pltpu.VMEM((NIDX,), jnp.int32),
            pltpu.VMEM((NIDX + L,), jnp.float32),
            pltpu.VMEM((NIDX,), jnp.int32),
            pltpu.VMEM((NIDX, C), jnp.float32),
            pltpu.VMEM((SB + L,), jnp.float32),
            pltpu.VMEM((SB, C), jnp.float32),
            pltpu.SemaphoreType.DMA,
        ],
    )
    def interp_kernel(xh, yh, zh, lut_hbm, data_hbm, out_hbm,
                      xv, yv, zv, keys_b, w_b, idx_b, rows_b, rinv_v,
                      out_v, sem):
        wid = lax.axis_index("s") * NC + lax.axis_index("c")

        def sub_batch(b, _):
            base = wid * npw + b * SB
            pltpu.sync_copy(xh.at[pl.ds(base, SB)], xv)
            pltpu.sync_copy(yh.at[pl.ds(base, SB)], yv)
            pltpu.sync_copy(zh.at[pl.ds(base, SB)], zv)

            def grp(g, _):
                o = g * L
                x = (xv[pl.ds(o, L)] + 1.0) * (GRID / 2) - 0.5
                y = (yv[pl.ds(o, L)] + 1.0) * (GRID / 2) - 0.5
                z = (zv[pl.ds(o, L)] + 1.0) * (GRID / 2) - 0.5
                xi, yi, zi = _floor_i32(x), _floor_i32(y), _floor_i32(z)
                fx = x - xi.astype(jnp.float32)
                fy = y - yi.astype(jnp.float32)
                fz = z - zi.astype(jnp.float32)
                px = (_interleave(xi, 2), _interleave(xi + 1, 2))
                py = (_interleave(yi, 1), _interleave(yi + 1, 1))
                pz = (_interleave(zi, 0), _interleave(zi + 1, 0))
                bx = ((xi >= 0) & (xi < GRID), (xi + 1 >= 0) & (xi + 1 < GRID))
                by = ((yi >= 0) & (yi < GRID), (yi + 1 >= 0) & (yi + 1 < GRID))
                bz = ((zi >= 0) & (zi < GRID), (zi + 1 >= 0) & (zi + 1 < GRID))
                wx = (1.0 - fx, -fx)
                wy = (1.0 - fy, -fy)
                wz = (1.0 - fz, -fz)
                for j in range(8):
                    cx, cy, cz = j >> 2, (j >> 1) & 1, j & 1
                    key = px[cx] | py[cy] | pz[cz]
                    bnd = bx[cx] & by[cy] & bz[cz]
                    keys_b[pl.ds(j * SB + o, L)] = jnp.where(
                        bnd, key, SENTINEL_KEY)
                    wgt = jnp.abs(wx[cx] * wy[cy] * wz[cz])
                    w_b[pl.ds(j * SB + o, L)] = jnp.where(bnd, wgt, 0.0)
                return 0

            lax.fori_loop(0, SB // L, grp, 0)

            cps = [pltpu.async_copy(
                lut_hbm.at[keys_b.at[pl.ds(d * DMA_I, DMA_I)]],
                idx_b.at[pl.ds(d * DMA_I, DMA_I)], sem)
                for d in range(NIDX // DMA_I)]
            for cp in cps:
                cp.wait()

            def fix(g, _):
                o = g * L
                s = jnp.zeros((L,), jnp.float32)
                for j in range(8):
                    iv = idx_b[pl.ds(j * SB + o, L)]
                    wv = jnp.where(iv == sentinel_node, 0.0,
                                   w_b[pl.ds(j * SB + o, L)])
                    w_b[pl.ds(j * SB + o, L)] = wv
                    s = s + wv
                rinv_v[pl.ds(o, L)] = 1.0 / (s + 1e-12)
                return 0

            lax.fori_loop(0, SB // L, fix, 0)

            cps = [pltpu.async_copy(
                data_hbm.at[idx_b.at[pl.ds(d * DMA_I, DMA_I)]],
                rows_b.at[pl.ds(d * DMA_I, DMA_I)], sem)
                for d in range(NIDX // DMA_I)]
            for cp in cps:
                cp.wait()

            def acc(p, _):
                a0 = jnp.zeros((L,), jnp.float32)
                a1 = jnp.zeros((L,), jnp.float32)
                for j in range(8):
                    r = j * SB + p
                    w = w_b[pl.ds(r, L)][0]
                    a0 = a0 + w * rows_b[r, pl.ds(0, L)]
                    a1 = a1 + w * rows_b[r, pl.ds(L, L)]
                rv = rinv_v[pl.ds(p, L)][0]
                out_v[p, pl.ds(0, L)] = a0 * rv
                out_v[p, pl.ds(L, L)] = a1 * rv
                return 0

            lax.fori_loop(0, SB, acc, 0)
            pltpu.sync_copy(out_v, out_hbm.at[pl.ds(base, SB)])
            return 0

        lax.fori_loop(0, nb, sub_batch, 0)

    return interp_kernel(xp, yp, zp, lut, data_aug)


def kernel(data, pts, node_keys, depth):
    n = pts.shape[0]
    nodes = node_keys.shape[0]
    nb = -(-n // (NW * SB))
    npw = nb * SB
    np_total = npw * NW

    xp = jnp.pad(pts[:, 0], (0, np_total - n))
    yp = jnp.pad(pts[:, 1], (0, np_total - n))
    zp = jnp.pad(pts[:, 2], (0, np_total - n))

    data_aug = jnp.concatenate(
        [data, jnp.zeros((8, data.shape[1]), jnp.float32)], axis=0)
    node_keys_pad = jnp.concatenate(
        [node_keys, jnp.full((NK_WIN,), PAD_KEY, jnp.int32)])
    bounds = jnp.arange(NW + L, dtype=jnp.int32) * LUT_SLICE
    starts8 = (jnp.searchsorted(node_keys, bounds).astype(jnp.int32)
               // 8) * 8
    starts8 = jnp.minimum(starts8, nodes)

    lut = _build_lut(node_keys_pad, starts8, nodes)
    out = _interp(xp, yp, zp, lut, data_aug, npw, nodes)
    return out[:n]


# speedup vs baseline: 55.9024x; 1.0014x over previous
import functools

import jax
import jax.numpy as jnp
from jax import lax
from jax.experimental import pallas as pl
from jax.experimental.pallas import tpu as pltpu
from jax.experimental.pallas import tpu_sc as plsc

NC = 2
NS = 16
NW = NC * NS
L = 16

DEPTH_C = 6
KEY_SPACE = 1 << (3 * DEPTH_C)
GRID = 1 << DEPTH_C
LUT_SLICE = 8224
LUT_SIZE = NW * LUT_SLICE
SENTINEL_KEY = KEY_SPACE
NK_WIN = LUT_SLICE + 16
PAD_KEY = 2 ** 30

SB = 256
NIDX = 8 * SB
DMA_I = 128
C = 32


def _interleave(v, s):
    k = jnp.zeros_like(v)
    for i in range(DEPTH_C):
        k = k | ((v & (1 << i)) << (2 * i + s))
    return k


def _floor_i32(xf):
    xt = xf.astype(jnp.int32)
    xi = jnp.where(xt.astype(jnp.float32) > xf, xt - 1, xt)
    return xi


def _build_lut(node_keys_pad, starts8, sentinel_node):
    mesh = plsc.VectorSubcoreMesh(core_axis_name="c", subcore_axis_name="s")

    @functools.partial(
        pl.kernel,
        out_type=jax.ShapeDtypeStruct((LUT_SIZE,), jnp.int32),
        mesh=mesh,
        compiler_params=pltpu.CompilerParams(
            needs_layout_passes=False, use_tc_tiling_on_sc=False),
        scratch_types=[
            pltpu.VMEM((NK_WIN,), jnp.int32),
            pltpu.VMEM((LUT_SLICE,), jnp.int32),
            pltpu.VMEM((NW + L,), jnp.int32),
        ],
    )
    def lut_kernel(nk_hbm, st_hbm, lut_hbm, nk_v, lut_v, st_v):
        wid = lax.axis_index("s") * NC + lax.axis_index("c")
        pltpu.sync_copy(st_hbm, st_v)
        s8 = pl.multiple_of(st_v[pl.ds(wid, L)][0], 8)
        pltpu.sync_copy(nk_hbm.at[pl.ds(s8, NK_WIN)], nk_v)

        fill = jnp.full((L,), sentinel_node, dtype=jnp.int32)

        def init_body(t, _):
            lut_v[pl.ds(t * L, L)] = fill
            return 0

        lax.fori_loop(0, LUT_SLICE // L, init_body, 0)

        base_local = wid * LUT_SLICE
        lanes = lax.iota(jnp.int32, L)

        def scat_body(t, _):
            kv = nk_v[pl.ds(t * L, L)]
            local = kv - base_local
            mask = (local >= 0) & (local < LUT_SLICE)
            localc = jnp.clip(local, 0, LUT_SLICE - 1)
            gi = s8 + t * L + lanes
            plsc.store_scatter(lut_v, [localc], gi, mask=mask)
            return 0

        lax.fori_loop(0, NK_WIN // L, scat_body, 0)
        pltpu.sync_copy(lut_v, lut_hbm.at[pl.ds(wid * LUT_SLICE, LUT_SLICE)])

    return lut_kernel(node_keys_pad, starts8)


def _interp(xp, yp, zp, lut, data_aug, npw, sentinel_node):
    mesh = plsc.VectorSubcoreMesh(core_axis_name="c", subcore_axis_name="s")
    np_total = npw * NW
    nb = npw // SB

    @functools.partial(
        pl.kernel,
        out_type=jax.ShapeDtypeStruct((np_total, C), jnp.float32),
        mesh=mesh,
        compiler_params=pltpu.CompilerParams(
            needs_layout_passes=False, use_tc_tiling_on_sc=False),
        scratch_types=[
            pltpu.VMEM((SB,), jnp.float32),
            pltpu.VMEM((SB,), jnp.float32),
            pltpu.VMEM((SB,), jnp.float32),
            pltpu.VMEM((NIDX,), jnp.int32),
            pltpu.VMEM((NIDX + L,), jnp.float32),
            pltpu.VMEM((NIDX,), jnp.int32),
            pltpu.VMEM((NIDX, C), jnp.float32),
            pltpu.VMEM((SB + L,), jnp.float32),
            pltpu.VMEM((SB, C), jnp.float32),
            pltpu.SemaphoreType.DMA,
            pltpu.SemaphoreType.DMA,
            pltpu.SemaphoreType.DMA,
            pltpu.SemaphoreType.DMA,
            pltpu.SemaphoreType.DMA,
        ],
    )
    def interp_kernel(xh, yh, zh, lut_hbm, data_hbm, out_hbm,
                      xv, yv, zv, keys_b, w_b, idx_b, rows_b, rinv_v,
                      out_v, sem, rs0, rs1, rs2, rs3):
        rsem = (rs0, rs1, rs2, rs3)
        wid = lax.axis_index("s") * NC + lax.axis_index("c")

        def sub_batch(b, _):
            base = wid * npw + b * SB
            pltpu.sync_copy(xh.at[pl.ds(base, SB)], xv)
            pltpu.sync_copy(yh.at[pl.ds(base, SB)], yv)
            pltpu.sync_copy(zh.at[pl.ds(base, SB)], zv)

            def grp(g, _):
                o = g * L
                x = (xv[pl.ds(o, L)] + 1.0) * (GRID / 2) - 0.5
                y = (yv[pl.ds(o, L)] + 1.0) * (GRID / 2) - 0.5
                z = (zv[pl.ds(o, L)] + 1.0) * (GRID / 2) - 0.5
                xi, yi, zi = _floor_i32(x), _floor_i32(y), _floor_i32(z)
                fx = x - xi.astype(jnp.float32)
                fy = y - yi.astype(jnp.float32)
                fz = z - zi.astype(jnp.float32)
                px = (_interleave(xi, 2), _interleave(xi + 1, 2))
                py = (_interleave(yi, 1), _interleave(yi + 1, 1))
                pz = (_interleave(zi, 0), _interleave(zi + 1, 0))
                bx = ((xi >= 0) & (xi < GRID), (xi + 1 >= 0) & (xi + 1 < GRID))
                by = ((yi >= 0) & (yi < GRID), (yi + 1 >= 0) & (yi + 1 < GRID))
                bz = ((zi >= 0) & (zi < GRID), (zi + 1 >= 0) & (zi + 1 < GRID))
                wx = (1.0 - fx, -fx)
                wy = (1.0 - fy, -fy)
                wz = (1.0 - fz, -fz)
                for j in range(8):
                    cx, cy, cz = j >> 2, (j >> 1) & 1, j & 1
                    key = px[cx] | py[cy] | pz[cz]
                    bnd = bx[cx] & by[cy] & bz[cz]
                    keys_b[pl.ds(j * SB + o, L)] = jnp.where(
                        bnd, key, SENTINEL_KEY)
                    wgt = jnp.abs(wx[cx] * wy[cy] * wz[cz])
                    w_b[pl.ds(j * SB + o, L)] = jnp.where(bnd, wgt, 0.0)
                return 0

            lax.fori_loop(0, SB // L, grp, 0)

            cps = [pltpu.async_copy(
                lut_hbm.at[keys_b.at[pl.ds(d * DMA_I, DMA_I)]],
                idx_b.at[pl.ds(d * DMA_I, DMA_I)], sem)
                for d in range(NIDX // DMA_I)]
            for cp in cps:
                cp.wait()

            def fix(g, _):
                o = g * L
                s = jnp.zeros((L,), jnp.float32)
                for j in range(8):
                    iv = idx_b[pl.ds(j * SB + o, L)]
                    wv = jnp.where(iv == sentinel_node, 0.0,
                                   w_b[pl.ds(j * SB + o, L)])
                    w_b[pl.ds(j * SB + o, L)] = wv
                    s = s + wv
                rinv_v[pl.ds(o, L)] = 1.0 / (s + 1e-12)
                return 0

            lax.fori_loop(0, SB // L, fix, 0)

            def zero(p, _):
                z = jnp.zeros((L,), jnp.float32)
                out_v[p, pl.ds(0, L)] = z
                out_v[p, pl.ds(L, L)] = z
                return 0

            lax.fori_loop(0, SB, zero, 0)

            def fire_row(d):
                return pltpu.async_copy(
                    data_hbm.at[idx_b.at[pl.ds(d * DMA_I, DMA_I)]],
                    rows_b.at[pl.ds(d * DMA_I, DMA_I)], rsem[d % 4])

            def acc_chunk(d):
                po = (d % 2) * DMA_I

                def acc(p, _):
                    r = d * DMA_I + p
                    pp = po + p
                    w = w_b[pl.ds(r, L)][0]
                    out_v[pp, pl.ds(0, L)] = (
                        out_v[pp, pl.ds(0, L)] + w * rows_b[r, pl.ds(0, L)])
                    out_v[pp, pl.ds(L, L)] = (
                        out_v[pp, pl.ds(L, L)] + w * rows_b[r, pl.ds(L, L)])
                    return 0

                lax.fori_loop(0, DMA_I, acc, 0)

            nd = NIDX // DMA_I
            row_cp = {d: fire_row(d) for d in range(4)}
            for d in range(nd):
                row_cp[d].wait()
                if d + 4 < nd:
                    row_cp[d + 4] = fire_row(d + 4)
                acc_chunk(d)

            def scale(p, _):
                rv = rinv_v[pl.ds(p, L)][0]
                out_v[p, pl.ds(0, L)] = out_v[p, pl.ds(0, L)] * rv
                out_v[p, pl.ds(L, L)] = out_v[p, pl.ds(L, L)] * rv
                return 0

            lax.fori_loop(0, SB, scale, 0)
            pltpu.sync_copy(out_v, out_hbm.at[pl.ds(base, SB)])
            return 0

        lax.fori_loop(0, nb, sub_batch, 0)

    return interp_kernel(xp, yp, zp, lut, data_aug)


def kernel(data, pts, node_keys, depth):
    n = pts.shape[0]
    nodes = node_keys.shape[0]
    nb = -(-n // (NW * SB))
    npw = nb * SB
    np_total = npw * NW

    xp = jnp.pad(pts[:, 0], (0, np_total - n))
    yp = jnp.pad(pts[:, 1], (0, np_total - n))
    zp = jnp.pad(pts[:, 2], (0, np_total - n))

    data_aug = jnp.concatenate(
        [data, jnp.zeros((8, data.shape[1]), jnp.float32)], axis=0)
    node_keys_pad = jnp.concatenate(
        [node_keys, jnp.full((NK_WIN,), PAD_KEY, jnp.int32)])
    bounds = jnp.arange(NW + L, dtype=jnp.int32) * LUT_SLICE
    starts8 = (jnp.searchsorted(node_keys, bounds).astype(jnp.int32)
               // 8) * 8
    starts8 = jnp.minimum(starts8, nodes)

    lut = _build_lut(node_keys_pad, starts8, nodes)
    out = _interp(xp, yp, zp, lut, data_aug, npw, nodes)
    return out[:n]
